# baseline (speedup 1.0000x reference)
import jax
import jax.numpy as jnp
from jax.experimental import pallas as pl
from jax.experimental.pallas import tpu as pltpu

_TN = 512


def _chamfer_kernel(p_ref, t_ref, loss_ref, runmin_ref):
    b = pl.program_id(0)
    n = pl.program_id(1)
    nt = pl.num_programs(1)
    inv_bn = 1.0 / (pl.num_programs(0) * nt * _TN)
    m = t_ref.shape[1]
    inv_bm = 1.0 / (pl.num_programs(0) * m)

    p = p_ref[0]
    t = t_ref[0]
    p2 = jnp.sum(p * p, axis=1, keepdims=True)
    t2 = jnp.sum(t * t, axis=1)[None, :]

    pm2 = (p * -2.0).astype(jnp.bfloat16)
    tb = t.astype(jnp.bfloat16)
    inner = jax.lax.dot_general(
        pm2, tb, (((1,), (1,)), ((), ())),
        preferred_element_type=jnp.float32)

    row_min = jnp.min(inner + t2, axis=1, keepdims=True)
    cham_x = jnp.maximum(row_min + p2, 0.0)

    @pl.when(jnp.logical_and(b == 0, n == 0))
    def _():
        loss_ref[0, 0] = 0.0

    loss_ref[0, 0] += jnp.sum(cham_x) * inv_bn

    col_min = jnp.min(inner + p2, axis=0, keepdims=True)

    @pl.when(n == 0)
    def _():
        runmin_ref[...] = col_min

    @pl.when(n != 0)
    def _():
        runmin_ref[...] = jnp.minimum(runmin_ref[...], col_min)

    @pl.when(n == nt - 1)
    def _():
        cham_y = jnp.maximum(runmin_ref[...] + t2, 0.0)
        loss_ref[0, 0] += jnp.sum(cham_y) * inv_bm


def kernel(pred, target):
    bsz, n, d = pred.shape
    m = target.shape[1]
    out = pl.pallas_call(
        _chamfer_kernel,
        grid=(bsz, n // _TN),
        in_specs=[
            pl.BlockSpec((1, _TN, d), lambda b, i: (b, i, 0)),
            pl.BlockSpec((1, m, d), lambda b, i: (b, 0, 0)),
        ],
        out_specs=pl.BlockSpec(
            (1, 1), lambda b, i: (0, 0), memory_space=pltpu.SMEM),
        out_shape=jax.ShapeDtypeStruct((1, 1), jnp.float32),
        scratch_shapes=[pltpu.VMEM((1, m), jnp.float32)],
        compiler_params=pltpu.CompilerParams(
            dimension_semantics=("arbitrary", "arbitrary")),
    )(pred, target)
    return out[0, 0]


# speedup vs baseline: 1.3532x; 1.3532x over previous
import jax
import jax.numpy as jnp
from jax.experimental import pallas as pl
from jax.experimental.pallas import tpu as pltpu

_TN = 512


def _chamfer_kernel(p_ref, t_ref, loss_ref, runmin_ref):
    b = pl.program_id(0)
    n = pl.program_id(1)
    nt = pl.num_programs(1)
    inv_bn = 1.0 / (pl.num_programs(0) * nt * _TN)
    m = t_ref.shape[1]
    inv_bm = 1.0 / (pl.num_programs(0) * m)

    p = p_ref[0]
    t = t_ref[0]
    p2 = jnp.sum(p * p, axis=1, keepdims=True)
    t2 = jnp.sum(t * t, axis=1, keepdims=True)

    ones_p = jnp.ones((p.shape[0], 1), jnp.bfloat16)
    ones_t = jnp.ones((t.shape[0], 1), jnp.bfloat16)
    a_aug = jnp.concatenate(
        [(p * -2.0).astype(jnp.bfloat16), p2.astype(jnp.bfloat16), ones_p],
        axis=1)
    b_aug = jnp.concatenate(
        [t.astype(jnp.bfloat16), ones_t, t2.astype(jnp.bfloat16)],
        axis=1)
    dist = jax.lax.dot_general(
        a_aug, b_aug, (((1,), (1,)), ((), ())),
        preferred_element_type=jnp.float32)

    cham_x = jnp.maximum(jnp.min(dist, axis=1, keepdims=True), 0.0)

    @pl.when(jnp.logical_and(b == 0, n == 0))
    def _():
        loss_ref[0, 0] = 0.0

    loss_ref[0, 0] += jnp.sum(cham_x) * inv_bn

    col_min = jnp.min(dist, axis=0, keepdims=True)

    @pl.when(n == 0)
    def _():
        runmin_ref[...] = col_min

    @pl.when(n != 0)
    def _():
        runmin_ref[...] = jnp.minimum(runmin_ref[...], col_min)

    @pl.when(n == nt - 1)
    def _():
        cham_y = jnp.maximum(runmin_ref[...], 0.0)
        loss_ref[0, 0] += jnp.sum(cham_y) * inv_bm


def kernel(pred, target):
    bsz, n, d = pred.shape
    m = target.shape[1]
    out = pl.pallas_call(
        _chamfer_kernel,
        grid=(bsz, n // _TN),
        in_specs=[
            pl.BlockSpec((1, _TN, d), lambda b, i: (b, i, 0)),
            pl.BlockSpec((1, m, d), lambda b, i: (b, 0, 0)),
        ],
        out_specs=pl.BlockSpec(
            (1, 1), lambda b, i: (0, 0), memory_space=pltpu.SMEM),
        out_shape=jax.ShapeDtypeStruct((1, 1), jnp.float32),
        scratch_shapes=[pltpu.VMEM((1, m), jnp.float32)],
        compiler_params=pltpu.CompilerParams(
            dimension_semantics=("arbitrary", "arbitrary")),
    )(pred, target)
    return out[0, 0]
